# baseline (speedup 1.0000x reference)
import functools

import jax
import jax.numpy as jnp
from jax import lax
from jax.experimental import pallas as pl
from jax.experimental.pallas import tpu as pltpu
from jax.experimental.pallas import tpu_sc as plsc

BATCH = 16384
EMB_DIM = 64
_INFO = plsc.get_sparse_core_info()
_NC, _NS, _L = _INFO.num_cores, _INFO.num_subcores, _INFO.num_lanes
_NW = _NC * _NS
_BPW = BATCH // _NW
_CHUNK = 128
_NCHUNK = _BPW // _CHUNK
_GROUPS = _BPW // _L


def _body(uid_hbm, bid_hbm, utab_hbm, btab_hbm, out_hbm,
          uid_v, bid_v, urows_v, brows_v, out_v, usem, bsem):
    wid = lax.axis_index("s") * _NC + lax.axis_index("c")
    base = wid * _BPW

    pltpu.sync_copy(uid_hbm.at[pl.ds(wid * _NCHUNK, _NCHUNK)], uid_v)
    pltpu.sync_copy(bid_hbm.at[pl.ds(wid * _NCHUNK, _NCHUNK)], bid_v)

    copies = []
    for j in range(_NCHUNK):
        copies.append(pltpu.async_copy(
            utab_hbm.at[uid_v.at[j]],
            urows_v.at[pl.ds(j * _CHUNK, _CHUNK)], usem))
        copies.append(pltpu.async_copy(
            btab_hbm.at[bid_v.at[j]],
            brows_v.at[pl.ds(j * _CHUNK, _CHUNK)], bsem))
    for c in copies:
        c.wait()

    lane = lax.iota(jnp.int32, _L)

    def group(g, _):
        row = g * _L + lane

        def step(d, acc):
            col = lax.bitwise_and(d + lane, EMB_DIM - 1)
            u = plsc.load_gather(urows_v, [row, col])
            b = plsc.load_gather(brows_v, [row, col])
            return acc + u * b

        acc = lax.fori_loop(0, EMB_DIM, step, jnp.zeros((_L,), jnp.float32))
        out_v[pl.ds(g * _L, _L)] = acc
        return 0

    lax.fori_loop(0, _GROUPS, group, 0)
    pltpu.sync_copy(out_v, out_hbm.at[pl.ds(base, _BPW)])


@jax.jit
def _run(uid2d, bid2d, user_table, banner_table):
    mesh = plsc.VectorSubcoreMesh(core_axis_name="c", subcore_axis_name="s")
    return pl.kernel(
        _body,
        mesh=mesh,
        compiler_params=pltpu.CompilerParams(
            needs_layout_passes=False, use_tc_tiling_on_sc=False),
        out_type=jax.ShapeDtypeStruct((BATCH,), jnp.float32),
        scratch_types=[
            pltpu.VMEM((_NCHUNK, _CHUNK), jnp.int32),
            pltpu.VMEM((_NCHUNK, _CHUNK), jnp.int32),
            pltpu.VMEM((_BPW, EMB_DIM), jnp.float32),
            pltpu.VMEM((_BPW, EMB_DIM), jnp.float32),
            pltpu.VMEM((_BPW,), jnp.float32),
            pltpu.SemaphoreType.DMA,
            pltpu.SemaphoreType.DMA,
        ],
    )(uid2d, bid2d, user_table, banner_table)


def kernel(user_ids, banner_ids, user_table, banner_table):
    uid2d = user_ids.astype(jnp.int32).reshape(_NW * _NCHUNK, _CHUNK)
    bid2d = banner_ids.astype(jnp.int32).reshape(_NW * _NCHUNK, _CHUNK)
    return _run(uid2d, bid2d, user_table, banner_table)


# speedup vs baseline: 2.0282x; 2.0282x over previous
import jax
import jax.numpy as jnp
from jax import lax
from jax.experimental import pallas as pl
from jax.experimental.pallas import tpu as pltpu
from jax.experimental.pallas import tpu_sc as plsc

BATCH = 16384
EMB_DIM = 64
_INFO = plsc.get_sparse_core_info()
_NC, _NS, _L = _INFO.num_cores, _INFO.num_subcores, _INFO.num_lanes
_NW = _NC * _NS
_BPW = BATCH // _NW
_NCHUNK = _BPW // _L


def _body(uid_hbm, bid_hbm, utab_hbm, btab_hbm, out_hbm,
          uid_v, bid_v, ub0, ub1, bb0, bb1, out_v,
          us0, us1, bs0, bs1):
    wid = lax.axis_index("s") * _NC + lax.axis_index("c")
    base = wid * _BPW

    pltpu.sync_copy(uid_hbm.at[pl.ds(base, _BPW)], uid_v)
    pltpu.sync_copy(bid_hbm.at[pl.ds(base, _BPW)], bid_v)

    ubufs, bbufs = (ub0, ub1), (bb0, bb1)
    usems, bsems = (us0, us1), (bs0, bs1)
    lane = lax.iota(jnp.int32, _L)

    def ids(c):
        return uid_v[pl.ds(c * _L, _L)], bid_v[pl.ds(c * _L, _L)]

    def compute(c, k):
        uvec, bvec = ids(c)
        urow = uvec & 7
        brow = bvec & 7

        def step(d, acc):
            col = lax.bitwise_and(d + lane, EMB_DIM - 1)
            u = plsc.load_gather(ubufs[k], [lane, urow, col])
            b = plsc.load_gather(bbufs[k], [lane, brow, col])
            return acc + u * b

        acc = lax.fori_loop(0, EMB_DIM, step, jnp.zeros((_L,), jnp.float32))
        out_v[pl.ds(c * _L, _L)] = acc

    def chunk(c, _):
        k = 0
        uvec, bvec = ids(c)
        ublk, bblk = uvec >> 3, bvec >> 3
        copies = []
        for j in range(_L):
            copies.append(pltpu.async_copy(
                utab_hbm.at[ublk[j]], ubufs[k].at[j], usems[k]))
            copies.append(pltpu.async_copy(
                btab_hbm.at[bblk[j]], bbufs[k].at[j], bsems[k]))
        for cp in copies:
            cp.wait()
        compute(c, k)
        return 0

    lax.fori_loop(0, _NCHUNK, chunk, 0)

    pltpu.sync_copy(out_v, out_hbm.at[pl.ds(base, _BPW)])


@jax.jit
def _run(uid, bid, utab3, btab3):
    mesh = plsc.VectorSubcoreMesh(core_axis_name="c", subcore_axis_name="s")
    return pl.kernel(
        _body,
        mesh=mesh,
        compiler_params=pltpu.CompilerParams(needs_layout_passes=False),
        out_type=jax.ShapeDtypeStruct((BATCH,), jnp.float32),
        scratch_types=[
            pltpu.VMEM((_BPW,), jnp.int32),
            pltpu.VMEM((_BPW,), jnp.int32),
            pltpu.VMEM((_L, 8, EMB_DIM), jnp.float32),
            pltpu.VMEM((_L, 8, EMB_DIM), jnp.float32),
            pltpu.VMEM((_L, 8, EMB_DIM), jnp.float32),
            pltpu.VMEM((_L, 8, EMB_DIM), jnp.float32),
            pltpu.VMEM((_BPW,), jnp.float32),
            pltpu.SemaphoreType.DMA,
            pltpu.SemaphoreType.DMA,
            pltpu.SemaphoreType.DMA,
            pltpu.SemaphoreType.DMA,
        ],
    )(uid, bid, utab3, btab3)


def kernel(user_ids, banner_ids, user_table, banner_table):
    utab3 = user_table.reshape(-1, 8, EMB_DIM)
    btab3 = banner_table.reshape(-1, 8, EMB_DIM)
    return _run(user_ids.astype(jnp.int32), banner_ids.astype(jnp.int32),
                utab3, btab3)


# speedup vs baseline: 2.0913x; 1.0311x over previous
import jax
import jax.numpy as jnp
from jax import lax
from jax.experimental import pallas as pl
from jax.experimental.pallas import tpu as pltpu
from jax.experimental.pallas import tpu_sc as plsc

BATCH = 16384
EMB_DIM = 64
_INFO = plsc.get_sparse_core_info()
_NC, _NS, _L = _INFO.num_cores, _INFO.num_subcores, _INFO.num_lanes
_NW = _NC * _NS
_BPW = BATCH // _NW
_NCHUNK = _BPW // _L
_DEPTH = 3


def _body(uid_hbm, bid_hbm, utab_hbm, btab_hbm, out_hbm,
          uid_v, bid_v, ub0, ub1, ub2, bb0, bb1, bb2, out_v,
          us0, us1, us2, bs0, bs1, bs2):
    wid = lax.axis_index("s") * _NC + lax.axis_index("c")
    base = wid * _BPW

    pltpu.sync_copy(uid_hbm.at[pl.ds(base, _BPW)], uid_v)
    pltpu.sync_copy(bid_hbm.at[pl.ds(base, _BPW)], bid_v)

    ubufs, bbufs = (ub0, ub1, ub2), (bb0, bb1, bb2)
    usems, bsems = (us0, us1, us2), (bs0, bs1, bs2)
    lane = lax.iota(jnp.int32, _L)

    def ids(c):
        return uid_v[pl.ds(c * _L, _L)], bid_v[pl.ds(c * _L, _L)]

    def compute(c, k):
        uvec, bvec = ids(c)
        urow = uvec & 7
        brow = bvec & 7

        def step(d, acc):
            col = lax.bitwise_and(d + lane, EMB_DIM - 1)
            u = plsc.load_gather(ubufs[k], [lane, urow, col])
            b = plsc.load_gather(bbufs[k], [lane, brow, col])
            return acc + u * b

        acc = lax.fori_loop(0, EMB_DIM, step, jnp.zeros((_L,), jnp.float32))
        out_v[pl.ds(c * _L, _L)] = acc

    def fire(c, k):
        uvec, bvec = ids(c)
        ublk, bblk = uvec >> 3, bvec >> 3
        copies = []
        for j in range(_L):
            copies.append(pltpu.async_copy(
                utab_hbm.at[ublk[j]], ubufs[k].at[j], usems[k]))
            copies.append(pltpu.async_copy(
                btab_hbm.at[bblk[j]], bbufs[k].at[j], bsems[k]))
        return copies

    def triple(t, nfire):
        c0 = t * _DEPTH
        fired = [fire(c0 + s, s) for s in range(nfire)]
        for s in range(nfire):
            for cp in fired[s]:
                cp.wait()
            compute(c0 + s, s)
        return 0

    lax.fori_loop(0, _NCHUNK // _DEPTH, lambda t, x: triple(t, _DEPTH), 0)
    if _NCHUNK % _DEPTH:
        triple(_NCHUNK // _DEPTH, _NCHUNK % _DEPTH)

    pltpu.sync_copy(out_v, out_hbm.at[pl.ds(base, _BPW)])


@jax.jit
def _run(uid, bid, utab3, btab3):
    mesh = plsc.VectorSubcoreMesh(core_axis_name="c", subcore_axis_name="s")
    return pl.kernel(
        _body,
        mesh=mesh,
        compiler_params=pltpu.CompilerParams(needs_layout_passes=False),
        out_type=jax.ShapeDtypeStruct((BATCH,), jnp.float32),
        scratch_types=[
            pltpu.VMEM((_BPW,), jnp.int32),
            pltpu.VMEM((_BPW,), jnp.int32),
            pltpu.VMEM((_L, 8, EMB_DIM), jnp.float32),
            pltpu.VMEM((_L, 8, EMB_DIM), jnp.float32),
            pltpu.VMEM((_L, 8, EMB_DIM), jnp.float32),
            pltpu.VMEM((_L, 8, EMB_DIM), jnp.float32),
            pltpu.VMEM((_L, 8, EMB_DIM), jnp.float32),
            pltpu.VMEM((_L, 8, EMB_DIM), jnp.float32),
            pltpu.VMEM((_BPW,), jnp.float32),
            pltpu.SemaphoreType.DMA,
            pltpu.SemaphoreType.DMA,
            pltpu.SemaphoreType.DMA,
            pltpu.SemaphoreType.DMA,
            pltpu.SemaphoreType.DMA,
            pltpu.SemaphoreType.DMA,
        ],
    )(uid, bid, utab3, btab3)


def kernel(user_ids, banner_ids, user_table, banner_table):
    utab3 = user_table.reshape(-1, 8, EMB_DIM)
    btab3 = banner_table.reshape(-1, 8, EMB_DIM)
    return _run(user_ids.astype(jnp.int32), banner_ids.astype(jnp.int32),
                utab3, btab3)
